# baseline (speedup 1.0000x reference)
import functools

import jax
import jax.numpy as jnp
from jax import lax
from jax.experimental import pallas as pl
from jax.experimental.pallas import tpu as pltpu
from jax.experimental.pallas import tpu_sc as plsc

B = 128
C = 112
P = 256
D = 10
NC = 200
MAX_DIST = 128.0
ALPHA = 1.0
COEFS = {"crs_ent": 1.0, "clst": 0.8, "sep": -0.08, "l1": 1e-4,
         "ortho_p": 1.0, "ortho_c": 1.0}
EPS = 1e-8
R2 = 2 * C
L = 16
NW = 32
RPW = R2 // NW
NBLK = P // L
NBB = B // L
BIGF = 3.0e38
NEGF = -3.0e38


def _sc_dist_body(mdt_hbm, ppf_hbm, wt_hbm, out1_hbm, out2_hbm,
                  mdv, ppv, wv, sv, a1v, a2v, tix, qv, sem_a, sem_m):
    cid = lax.axis_index("c")
    sid = lax.axis_index("s")
    wid = sid * 2 + cid
    cps = []
    for d in range(D):
        cps.append(pltpu.async_copy(
            ppf_hbm.at[pl.ds(d * (R2 * P) + wid * (RPW * P), RPW * P)],
            ppv.at[pl.ds(d * (RPW * P), RPW * P)], sem_a))
    cps.append(pltpu.async_copy(
        wt_hbm.at[pl.ds(wid * RPW * B, RPW * B)], wv, sem_a))
    cpm = pltpu.async_copy(mdt_hbm, mdv, sem_m)
    for cp in cps:
        cp.wait()

    iota = lax.iota(jnp.int32, L)

    def srow_body(i, _):
        pbase = i * P
        s_chunks = []
        for blk in range(NBLK):
            off = pbase + blk * L
            acc = ppv[pl.ds(off, L)]
            for d in range(1, D):
                acc = acc + ppv[pl.ds(d * (RPW * P) + off, L)]
            s_chunks.append(acc)

        for j in range(D):
            m = s_chunks[0]
            for blk in range(1, NBLK):
                m = jnp.maximum(m, s_chunks[blk])
            gm = jnp.max(m)
            cand = jnp.where(s_chunks[0] == gm, iota, jnp.int32(P))
            for blk in range(1, NBLK):
                cblk = jnp.where(s_chunks[blk] == gm, iota + blk * L,
                                 jnp.int32(P))
                cand = jnp.minimum(cand, cblk)
            idx = jnp.min(cand)
            tix[i * L + j] = idx
            for blk in range(NBLK):
                hit = (iota + blk * L) == idx
                s_chunks[blk] = jnp.where(hit, jnp.float32(NEGF),
                                          s_chunks[blk])
        return 0
    lax.fori_loop(0, RPW, srow_body, 0)

    cpm.wait()

    def qpre_body(n, _):
        k = n // (P // 16)
        t0 = (n - k * (P // 16)) * 4
        b0 = k * L
        for u in range(4):
            t = t0 + u
            x = jnp.minimum(mdv[4 * t, pl.ds(b0, L)],
                            mdv[4 * t + 1, pl.ds(b0, L)])
            y = jnp.minimum(mdv[4 * t + 2, pl.ds(b0, L)],
                            mdv[4 * t + 3, pl.ds(b0, L)])
            qv[k * (P // 4) + t, :] = jnp.minimum(x, y)
        return 0
    lax.fori_loop(0, NBB * (P // 16), qpre_body, 0)

    def row_body(i, accs):
        acc1, acc2 = accs
        tj = [tix[i * L + j] for j in range(D)]
        tq = [lax.shift_right_logical(t, 2) for t in tj]

        def patch_body(k, _):
            b0 = k * L
            for j in range(D):
                sv[j, pl.ds(b0, L)] = mdv[tj[j], pl.ds(b0, L)]
                mdv[tj[j], pl.ds(b0, L)] = jnp.full((L,), BIGF, jnp.float32)
            return 0
        lax.fori_loop(0, NBB, patch_body, 0)

        def qpatch_body(k, _):
            b0 = k * L
            for j in range(D):
                x = jnp.minimum(mdv[4 * tq[j], pl.ds(b0, L)],
                                mdv[4 * tq[j] + 1, pl.ds(b0, L)])
                y = jnp.minimum(mdv[4 * tq[j] + 2, pl.ds(b0, L)],
                                mdv[4 * tq[j] + 3, pl.ds(b0, L)])
                qv[k * (P // 4) + tq[j], :] = jnp.minimum(x, y)
            return 0
        lax.fori_loop(0, NBB, qpatch_body, 0)

        def bb_body(k, accs2):
            a1, a2 = accs2
            b0 = k * L
            big = jnp.full((L,), BIGF, jnp.float32)
            q0 = k * (P // 4)

            def p_body(q, c):
                t0 = q0 + q * 16
                cs = list(c)
                for j in range(16):
                    cs[j % 8] = jnp.minimum(cs[j % 8], qv[t0 + j, :])
                return tuple(cs)
            cs = lax.fori_loop(0, P // 64, p_body, (big,) * 8)
            cs = list(cs)
            for st in (4, 2, 1):
                for j in range(st):
                    cs[j] = jnp.minimum(cs[j], cs[j + st])
            m2 = jnp.maximum(MAX_DIST - cs[0], 0.0)

            mt = sv[0, pl.ds(b0, L)]
            for j in range(1, D):
                mt = jnp.minimum(mt, sv[j, pl.ds(b0, L)])
            m1 = jnp.maximum(MAX_DIST - mt, 0.0)

            w = wv[pl.ds(i * B + b0, L)]
            return (a1 + w * m1, a2 + (1.0 - w) * m2)
        acc1, acc2 = lax.fori_loop(0, NBB, bb_body, (acc1, acc2))

        def rest_body(k, _):
            b0 = k * L
            for j in range(D):
                mdv[tj[j], pl.ds(b0, L)] = sv[j, pl.ds(b0, L)]
            for j in range(D):
                x = jnp.minimum(mdv[4 * tq[j], pl.ds(b0, L)],
                                mdv[4 * tq[j] + 1, pl.ds(b0, L)])
                y = jnp.minimum(mdv[4 * tq[j] + 2, pl.ds(b0, L)],
                                mdv[4 * tq[j] + 3, pl.ds(b0, L)])
                qv[k * (P // 4) + tq[j], :] = jnp.minimum(x, y)
            return 0
        lax.fori_loop(0, NBB, rest_body, 0)
        return (acc1, acc2)

    zero = jnp.zeros((L,), jnp.float32)
    acc1, acc2 = lax.fori_loop(0, RPW, row_body, (zero, zero))
    a1v[...] = acc1
    a2v[...] = acc2
    pltpu.sync_copy(a1v, out1_hbm.at[wid])
    pltpu.sync_copy(a2v, out2_hbm.at[wid])


_sc_dist = functools.partial(
    pl.kernel,
    mesh=plsc.VectorSubcoreMesh(core_axis_name="c", subcore_axis_name="s"),
    compiler_params=pltpu.CompilerParams(needs_layout_passes=False),
    out_type=[jax.ShapeDtypeStruct((NW, L), jnp.float32),
              jax.ShapeDtypeStruct((NW, L), jnp.float32)],
    scratch_types=[
        pltpu.VMEM((P, B), jnp.float32),
        pltpu.VMEM((RPW * P * D,), jnp.float32),
        pltpu.VMEM((RPW * B,), jnp.float32),
        pltpu.VMEM((D, B), jnp.float32),
        pltpu.VMEM((L,), jnp.float32),
        pltpu.VMEM((L,), jnp.float32),
        pltpu.SMEM((RPW * L,), jnp.int32),
        pltpu.VMEM((NBB * (P // 4), L), jnp.float32),
        pltpu.SemaphoreType.DMA,
        pltpu.SemaphoreType.DMA,
    ],
)(_sc_dist_body)


def _dense_body(cpl_ref, ctf_ref, eppt_ref, tpl_ref, tt_ref, w_ref, l1m_ref,
                part_ref, closs_ref, rn_ref):
    x = cpl_ref[...]
    ct = ctf_ref[...]
    bce = (jnp.maximum(x, 0.0) - x * ct
           + jnp.log(1.0 + jnp.exp(-jnp.abs(x))))
    closs_ref[...] = jnp.mean(bce, axis=0, keepdims=True)
    entropy = jnp.mean(bce)

    def rn_body(d, _):
        xd = eppt_ref[d]
        na = jnp.sqrt(jnp.sum(xd * xd, axis=1, keepdims=True))
        rn_ref[d] = 1.0 / jnp.maximum(na, EPS)
        return 0
    lax.fori_loop(0, D, rn_body, 0)

    def u_body(d, u):
        return u + eppt_ref[d] * rn_ref[d]
    u = lax.fori_loop(0, D, u_body, jnp.zeros((R2, P), jnp.float32))
    accp = jnp.sum(u * u)
    accc = jnp.sum(u[C:, :] * u[:C, :])
    ortho_p = accp / (D * C * 2) - 1.0
    ortho_c = accc / (D * C)

    tl = tpl_ref[...]
    m = jnp.max(tl, axis=1, keepdims=True)
    lse = m + jnp.log(jnp.sum(jnp.exp(tl - m), axis=1, keepdims=True))
    lbl = tt_ref[...]
    onehot = lax.broadcasted_iota(jnp.int32, (B, NC), 1) == lbl
    picked = jnp.sum(jnp.where(onehot, tl, 0.0), axis=1, keepdims=True)
    target_loss = jnp.mean(lse - picked)

    l1 = jnp.sum(jnp.abs(w_ref[...] * l1m_ref[...]))

    li = lax.broadcasted_iota(jnp.int32, (1, 16), 1)
    vals = [target_loss, entropy, l1, ortho_p, ortho_c]
    out = jnp.zeros((1, 16), jnp.float32)
    for i, val in enumerate(vals):
        out = jnp.where(li == i, val, out)
    part_ref[...] = out


def _combine_body(part_ref, acc1_ref, acc2_ref, scal_ref):
    target_loss = part_ref[0, 0]
    entropy = part_ref[0, 1]
    l1 = part_ref[0, 2]
    ortho_p = part_ref[0, 3]
    ortho_c = part_ref[0, 4]
    clst = MAX_DIST - jnp.sum(acc1_ref[...]) / (B * C)
    sep = MAX_DIST - jnp.sum(acc2_ref[...]) / (B * C)
    summed = (COEFS["crs_ent"] * entropy + COEFS["clst"] * clst
              + COEFS["sep"] * sep + COEFS["l1"] * l1
              + COEFS["ortho_p"] * ortho_p + COEFS["ortho_c"] * ortho_c)
    total = target_loss + ALPHA * summed

    li = lax.broadcasted_iota(jnp.int32, (1, 16), 1)
    vals = [target_loss, summed, total, entropy, clst, sep, l1,
            ortho_p, ortho_c]
    out = jnp.zeros((1, 16), jnp.float32)
    for i, val in enumerate(vals):
        out = jnp.where(li == i, val, out)
    scal_ref[...] = out


@jax.jit
def kernel(concepts_pred_logits, concepts_true, min_distances, proto_presence,
           target_pred_logits, target_true, encoder_proto_presence,
           last_layer_weight, l1_mask):
    ctf = concepts_true.astype(jnp.float32)
    eppt = jnp.transpose(encoder_proto_presence, (2, 0, 1))
    w1t = jnp.concatenate([ctf, 1.0 - ctf], axis=1).T.reshape(-1)
    mdt = min_distances.T
    ppf = jnp.transpose(proto_presence, (2, 0, 1)).reshape(-1)
    tt2 = target_true.astype(jnp.int32).reshape(B, 1)

    acc1, acc2 = _sc_dist(mdt, ppf, w1t)

    part, closs = pl.pallas_call(
        _dense_body,
        out_shape=[
            jax.ShapeDtypeStruct((1, 16), jnp.float32),
            jax.ShapeDtypeStruct((1, C), jnp.float32),
        ],
        scratch_shapes=[pltpu.VMEM((D, R2, 1), jnp.float32)],
    )(concepts_pred_logits, ctf, eppt, target_pred_logits, tt2,
      last_layer_weight, l1_mask)

    scal = pl.pallas_call(
        _combine_body,
        out_shape=jax.ShapeDtypeStruct((1, 16), jnp.float32),
    )(part, acc1, acc2)

    return (scal[0, 0], closs[0], scal[0, 1], scal[0, 2], scal[0, 3],
            scal[0, 4], scal[0, 5], scal[0, 6], scal[0, 7], scal[0, 8])


# speedup vs baseline: 1.1176x; 1.1176x over previous
import functools

import jax
import jax.numpy as jnp
from jax import lax
from jax.experimental import pallas as pl
from jax.experimental.pallas import tpu as pltpu
from jax.experimental.pallas import tpu_sc as plsc

B = 128
C = 112
P = 256
D = 10
NC = 200
MAX_DIST = 128.0
ALPHA = 1.0
COEFS = {"crs_ent": 1.0, "clst": 0.8, "sep": -0.08, "l1": 1e-4,
         "ortho_p": 1.0, "ortho_c": 1.0}
EPS = 1e-8
R2 = 2 * C
L = 16
NW = 32
RPW = R2 // NW
NBLK = P // L
NBB = B // L
BIGF = 3.0e38
NEGF = -3.0e38


def _sc_dist_body(mdt_hbm, ppf_hbm, wt_hbm, out1_hbm, out2_hbm,
                  mdv, ppv, wv, sv, a1v, a2v, tix, sem_a, sem_m):
    cid = lax.axis_index("c")
    sid = lax.axis_index("s")
    wid = sid * 2 + cid
    cps = []
    for d in range(D):
        cps.append(pltpu.async_copy(
            ppf_hbm.at[pl.ds(d * (R2 * P) + wid * (RPW * P), RPW * P)],
            ppv.at[pl.ds(d * (RPW * P), RPW * P)], sem_a))
    cps.append(pltpu.async_copy(
        wt_hbm.at[pl.ds(wid * RPW * B, RPW * B)], wv, sem_a))
    cpm = pltpu.async_copy(mdt_hbm, mdv, sem_m)
    for cp in cps:
        cp.wait()

    iota = lax.iota(jnp.int32, L)

    def srow_body(i, _):
        pbase = i * P
        s_chunks = []
        for blk in range(NBLK):
            off = pbase + blk * L
            acc = ppv[pl.ds(off, L)]
            for d in range(1, D):
                acc = acc + ppv[pl.ds(d * (RPW * P) + off, L)]
            s_chunks.append(acc)

        for j in range(D):
            m = s_chunks[0]
            for blk in range(1, NBLK):
                m = jnp.maximum(m, s_chunks[blk])
            gm = jnp.max(m)
            cand = jnp.where(s_chunks[0] == gm, iota, jnp.int32(P))
            for blk in range(1, NBLK):
                cblk = jnp.where(s_chunks[blk] == gm, iota + blk * L,
                                 jnp.int32(P))
                cand = jnp.minimum(cand, cblk)
            idx = jnp.min(cand)
            tix[i * L + j] = idx
            for blk in range(NBLK):
                hit = (iota + blk * L) == idx
                s_chunks[blk] = jnp.where(hit, jnp.float32(NEGF),
                                          s_chunks[blk])
        return 0
    lax.fori_loop(0, RPW, srow_body, 0)

    cpm.wait()

    def row_body(i, accs):
        acc1, acc2 = accs
        tj = [tix[i * L + j] for j in range(D)]

        def patch_body(k, _):
            b0 = k * L
            for j in range(D):
                sv[j, pl.ds(b0, L)] = mdv[tj[j], pl.ds(b0, L)]
                mdv[tj[j], pl.ds(b0, L)] = jnp.full((L,), BIGF, jnp.float32)
            return 0
        lax.fori_loop(0, NBB, patch_body, 0)

        def bb_body(k, accs2):
            a1, a2 = accs2
            b0 = k * L
            big = jnp.full((L,), BIGF, jnp.float32)

            def p_body(q, c):
                p0 = q * 16
                cs = list(c)
                for j in range(16):
                    cs[j % 8] = jnp.minimum(cs[j % 8],
                                            mdv[p0 + j, pl.ds(b0, L)])
                return tuple(cs)
            cs = lax.fori_loop(0, P // 16, p_body, (big,) * 8)
            cs = list(cs)
            for st in (4, 2, 1):
                for j in range(st):
                    cs[j] = jnp.minimum(cs[j], cs[j + st])
            m2 = jnp.maximum(MAX_DIST - cs[0], 0.0)

            mt = sv[0, pl.ds(b0, L)]
            for j in range(1, D):
                mt = jnp.minimum(mt, sv[j, pl.ds(b0, L)])
            m1 = jnp.maximum(MAX_DIST - mt, 0.0)

            w = wv[pl.ds(i * B + b0, L)]
            return (a1 + w * m1, a2 + (1.0 - w) * m2)
        acc1, acc2 = lax.fori_loop(0, NBB, bb_body, (acc1, acc2))

        def rest_body(k, _):
            b0 = k * L
            for j in range(D):
                mdv[tj[j], pl.ds(b0, L)] = sv[j, pl.ds(b0, L)]
            return 0
        lax.fori_loop(0, NBB, rest_body, 0)
        return (acc1, acc2)

    zero = jnp.zeros((L,), jnp.float32)
    acc1, acc2 = lax.fori_loop(0, RPW, row_body, (zero, zero))
    a1v[...] = acc1
    a2v[...] = acc2
    pltpu.sync_copy(a1v, out1_hbm.at[wid])
    pltpu.sync_copy(a2v, out2_hbm.at[wid])


_sc_dist = functools.partial(
    pl.kernel,
    mesh=plsc.VectorSubcoreMesh(core_axis_name="c", subcore_axis_name="s"),
    compiler_params=pltpu.CompilerParams(needs_layout_passes=False),
    out_type=[jax.ShapeDtypeStruct((NW, L), jnp.float32),
              jax.ShapeDtypeStruct((NW, L), jnp.float32)],
    scratch_types=[
        pltpu.VMEM((P, B), jnp.float32),
        pltpu.VMEM((RPW * P * D,), jnp.float32),
        pltpu.VMEM((RPW * B,), jnp.float32),
        pltpu.VMEM((D, B), jnp.float32),
        pltpu.VMEM((L,), jnp.float32),
        pltpu.VMEM((L,), jnp.float32),
        pltpu.SMEM((RPW * L,), jnp.int32),
        pltpu.SemaphoreType.DMA,
        pltpu.SemaphoreType.DMA,
    ],
)(_sc_dist_body)


def _dense_body(cpl_ref, ctf_ref, eppt_ref, tpl_ref, tt_ref, w_ref, l1m_ref,
                part_ref, closs_ref, rn_ref):
    x = cpl_ref[...]
    ct = ctf_ref[...]
    bce = (jnp.maximum(x, 0.0) - x * ct
           + jnp.log(1.0 + jnp.exp(-jnp.abs(x))))
    closs_ref[...] = jnp.mean(bce, axis=0, keepdims=True)
    entropy = jnp.mean(bce)

    def rn_body(d, _):
        xd = eppt_ref[d]
        na = jnp.sqrt(jnp.sum(xd * xd, axis=1, keepdims=True))
        rn_ref[d] = 1.0 / jnp.maximum(na, EPS)
        return 0
    lax.fori_loop(0, D, rn_body, 0)

    def u_body(d, u):
        return u + eppt_ref[d] * rn_ref[d]
    u = lax.fori_loop(0, D, u_body, jnp.zeros((R2, P), jnp.float32))
    accp = jnp.sum(u * u)
    accc = jnp.sum(u[C:, :] * u[:C, :])
    ortho_p = accp / (D * C * 2) - 1.0
    ortho_c = accc / (D * C)

    tl = tpl_ref[...]
    m = jnp.max(tl, axis=1, keepdims=True)
    lse = m + jnp.log(jnp.sum(jnp.exp(tl - m), axis=1, keepdims=True))
    lbl = tt_ref[...]
    onehot = lax.broadcasted_iota(jnp.int32, (B, NC), 1) == lbl
    picked = jnp.sum(jnp.where(onehot, tl, 0.0), axis=1, keepdims=True)
    target_loss = jnp.mean(lse - picked)

    l1 = jnp.sum(jnp.abs(w_ref[...] * l1m_ref[...]))

    li = lax.broadcasted_iota(jnp.int32, (1, 16), 1)
    vals = [target_loss, entropy, l1, ortho_p, ortho_c]
    out = jnp.zeros((1, 16), jnp.float32)
    for i, val in enumerate(vals):
        out = jnp.where(li == i, val, out)
    part_ref[...] = out


def _combine_body(part_ref, acc1_ref, acc2_ref, scal_ref):
    target_loss = part_ref[0, 0]
    entropy = part_ref[0, 1]
    l1 = part_ref[0, 2]
    ortho_p = part_ref[0, 3]
    ortho_c = part_ref[0, 4]
    clst = MAX_DIST - jnp.sum(acc1_ref[...]) / (B * C)
    sep = MAX_DIST - jnp.sum(acc2_ref[...]) / (B * C)
    summed = (COEFS["crs_ent"] * entropy + COEFS["clst"] * clst
              + COEFS["sep"] * sep + COEFS["l1"] * l1
              + COEFS["ortho_p"] * ortho_p + COEFS["ortho_c"] * ortho_c)
    total = target_loss + ALPHA * summed

    li = lax.broadcasted_iota(jnp.int32, (1, 16), 1)
    vals = [target_loss, summed, total, entropy, clst, sep, l1,
            ortho_p, ortho_c]
    out = jnp.zeros((1, 16), jnp.float32)
    for i, val in enumerate(vals):
        out = jnp.where(li == i, val, out)
    scal_ref[...] = out


@jax.jit
def kernel(concepts_pred_logits, concepts_true, min_distances, proto_presence,
           target_pred_logits, target_true, encoder_proto_presence,
           last_layer_weight, l1_mask):
    ctf = concepts_true.astype(jnp.float32)
    eppt = jnp.transpose(encoder_proto_presence, (2, 0, 1))
    w1t = jnp.concatenate([ctf, 1.0 - ctf], axis=1).T.reshape(-1)
    mdt = min_distances.T
    ppf = jnp.transpose(proto_presence, (2, 0, 1)).reshape(-1)
    tt2 = target_true.astype(jnp.int32).reshape(B, 1)

    acc1, acc2 = _sc_dist(mdt, ppf, w1t)

    part, closs = pl.pallas_call(
        _dense_body,
        out_shape=[
            jax.ShapeDtypeStruct((1, 16), jnp.float32),
            jax.ShapeDtypeStruct((1, C), jnp.float32),
        ],
        scratch_shapes=[pltpu.VMEM((D, R2, 1), jnp.float32)],
    )(concepts_pred_logits, ctf, eppt, target_pred_logits, tt2,
      last_layer_weight, l1_mask)

    scal = pl.pallas_call(
        _combine_body,
        out_shape=jax.ShapeDtypeStruct((1, 16), jnp.float32),
    )(part, acc1, acc2)

    return (scal[0, 0], closs[0], scal[0, 1], scal[0, 2], scal[0, 3],
            scal[0, 4], scal[0, 5], scal[0, 6], scal[0, 7], scal[0, 8])
